# Initial kernel scaffold; baseline (speedup 1.0000x reference)
import jax
import jax.numpy as jnp
from jax.experimental import pallas as pl


def kernel(x, edge_index, W_g, b_g, W_l, b_l):
    raise NotImplementedError("write your pallas kernel here")


# speedup vs baseline: 14.2946x; 14.2946x over previous
import functools

import jax
import jax.numpy as jnp
from jax import lax
from jax.experimental import pallas as pl
from jax.experimental.pallas import tpu as pltpu
from jax.experimental.pallas import tpu_sc as plsc

N_NODES = 10000
D_IN = 128
D_HID = 128
D_OUT = 64

NC, NS = 2, 16
NW = NC * NS
CH = 128
ROWS = 10112
STRIPE = ROWS // NS

TC_BLK = 2000
TC_GRID = N_NODES // TC_BLK


def _sc_mesh():
    return plsc.VectorSubcoreMesh(
        core_axis_name="c", subcore_axis_name="s", num_cores=NC, num_subcores=NS
    )


def _sc_degree(dst_p, kch):
    @functools.partial(
        pl.kernel,
        out_type=jax.ShapeDtypeStruct((NC, ROWS, 16), jnp.float32),
        mesh=_sc_mesh(),
        scratch_types=[
            pltpu.VMEM_SHARED((ROWS, 16), jnp.float32),
            pltpu.VMEM((kch, CH), jnp.int32),
            pltpu.VMEM((CH, 16), jnp.float32),
            pltpu.VMEM((STRIPE, 16), jnp.float32),
        ],
    )
    def k(dst_hbm, out_hbm, acc_sh, idx_v, ones_v, zero_v):
        cid = lax.axis_index("c")
        sid = lax.axis_index("s")
        wid = cid * NS + sid
        base = sid * STRIPE

        def fill_ones(i, carry):
            ones_v[i, :] = jnp.ones((16,), jnp.float32)
            return carry

        lax.fori_loop(0, CH, fill_ones, 0)

        def fill_zero(i, carry):
            zero_v[i, :] = jnp.zeros((16,), jnp.float32)
            return carry

        lax.fori_loop(0, STRIPE, fill_zero, 0)

        pltpu.sync_copy(zero_v, acc_sh.at[pl.ds(base, STRIPE)])
        plsc.subcore_barrier()

        pltpu.sync_copy(dst_hbm.at[wid], idx_v)

        def step(j, carry):
            pltpu.sync_copy(ones_v, acc_sh.at[idx_v.at[j]], add=True)
            return carry

        lax.fori_loop(0, kch, step, 0)
        plsc.subcore_barrier()
        pltpu.sync_copy(
            acc_sh.at[pl.ds(base, STRIPE)], out_hbm.at[cid, pl.ds(base, STRIPE)]
        )

    return k(dst_p)


def _sc_aggregate(y, src_p, dst_p, kch):
    ig = 16
    assert kch % ig == 0

    @functools.partial(
        pl.kernel,
        out_type=jax.ShapeDtypeStruct((NC, ROWS, D_HID), jnp.float32),
        mesh=_sc_mesh(),
        scratch_types=[
            pltpu.VMEM_SHARED((ROWS, D_HID), jnp.float32),
            pltpu.VMEM((ig, CH), jnp.int32),
            pltpu.VMEM((ig, CH), jnp.int32),
            pltpu.VMEM((CH, D_HID), jnp.float32),
            pltpu.SemaphoreType.DMA,
        ],
    )
    def k(y_hbm, src_hbm, dst_hbm, out_hbm, acc_sh, sidx, didx, rows_v, sem):
        cid = lax.axis_index("c")
        sid = lax.axis_index("s")
        wid = cid * NS + sid
        base = sid * STRIPE

        def fill_zero(i, carry):
            for c in range(D_HID // 16):
                rows_v[i, pl.ds(c * 16, 16)] = jnp.zeros((16,), jnp.float32)
            return carry

        lax.fori_loop(0, CH, fill_zero, 0)

        for off in range(0, STRIPE - CH + 1, CH):
            pltpu.sync_copy(rows_v, acc_sh.at[pl.ds(base + off, CH)])
        rem = STRIPE % CH
        if rem:
            pltpu.sync_copy(
                rows_v.at[pl.ds(0, rem)], acc_sh.at[pl.ds(base + STRIPE - rem, rem)]
            )
        plsc.subcore_barrier()

        def group(g, carry):
            pltpu.sync_copy(src_hbm.at[wid, pl.ds(g * ig, ig)], sidx)
            pltpu.sync_copy(dst_hbm.at[wid, pl.ds(g * ig, ig)], didx)

            def step(j, carry2):
                pltpu.async_copy(y_hbm.at[sidx.at[j]], rows_v, sem).wait()
                pltpu.sync_copy(rows_v, acc_sh.at[didx.at[j]], add=True)
                return carry2

            lax.fori_loop(0, ig, step, 0)
            return carry

        lax.fori_loop(0, kch // ig, group, 0)
        plsc.subcore_barrier()
        pltpu.sync_copy(
            acc_sh.at[pl.ds(base, STRIPE)], out_hbm.at[cid, pl.ds(base, STRIPE)]
        )

    return k(y, src_p, dst_p)


def _tc_y_body(degp_ref, x_ref, wg_ref, y_ref):
    dsum = degp_ref[0] + degp_ref[1]
    dinv = lax.rsqrt(dsum[:, 0:1] + 1.0)
    xw = jnp.dot(x_ref[...], wg_ref[...], preferred_element_type=jnp.float32)
    y_ref[...] = xw * dinv


def _tc_head_body(degp_ref, p_ref, y_ref, bg_ref, wl_ref, bl_ref, out_ref):
    dsum = degp_ref[0] + degp_ref[1]
    dinv = lax.rsqrt(dsum[:, 0:1] + 1.0)
    agg = p_ref[0] + p_ref[1] + y_ref[...]
    h = jnp.maximum(agg * dinv + bg_ref[...], 0.0)
    out_ref[...] = (
        jnp.dot(h, wl_ref[...], preferred_element_type=jnp.float32) + bl_ref[...]
    )


def kernel(x, edge_index, W_g, b_g, W_l, b_l):
    src = edge_index[0].astype(jnp.int32)
    dst = edge_index[1].astype(jnp.int32)
    n_edges = src.shape[0]
    per_w = n_edges // NW
    kch = -(-per_w // CH)
    kch = -(-kch // 16) * 16
    per_w_pad = kch * CH

    pad = per_w_pad - per_w
    src_p = jnp.concatenate(
        [src.reshape(NW, per_w), jnp.zeros((NW, pad), jnp.int32)], axis=1
    ).reshape(NW, kch, CH)
    garbage = N_NODES + (jnp.arange(pad, dtype=jnp.int32) % 16)
    dst_p = jnp.concatenate(
        [dst.reshape(NW, per_w), jnp.broadcast_to(garbage, (NW, pad))], axis=1
    ).reshape(NW, kch, CH)

    degp = _sc_degree(dst_p, kch)

    y = pl.pallas_call(
        _tc_y_body,
        grid=(TC_GRID,),
        in_specs=[
            pl.BlockSpec((NC, TC_BLK, 16), lambda i: (0, i, 0)),
            pl.BlockSpec((TC_BLK, D_IN), lambda i: (i, 0)),
            pl.BlockSpec((D_IN, D_HID), lambda i: (0, 0)),
        ],
        out_specs=pl.BlockSpec((TC_BLK, D_HID), lambda i: (i, 0)),
        out_shape=jax.ShapeDtypeStruct((N_NODES, D_HID), jnp.float32),
    )(degp, x, W_g)

    p = _sc_aggregate(y, src_p, dst_p, kch)

    out = pl.pallas_call(
        _tc_head_body,
        grid=(TC_GRID,),
        in_specs=[
            pl.BlockSpec((NC, TC_BLK, 16), lambda i: (0, i, 0)),
            pl.BlockSpec((NC, TC_BLK, D_HID), lambda i: (0, i, 0)),
            pl.BlockSpec((TC_BLK, D_HID), lambda i: (i, 0)),
            pl.BlockSpec((1, D_HID), lambda i: (0, 0)),
            pl.BlockSpec((D_HID, D_OUT), lambda i: (0, 0)),
            pl.BlockSpec((1, D_OUT), lambda i: (0, 0)),
        ],
        out_specs=pl.BlockSpec((TC_BLK, D_OUT), lambda i: (i, 0)),
        out_shape=jax.ShapeDtypeStruct((N_NODES, D_OUT), jnp.float32),
    )(degp, p, y, b_g.reshape(1, D_HID), W_l, b_l.reshape(1, D_OUT))

    return out


# speedup vs baseline: 16.0318x; 1.1215x over previous
import functools

import jax
import jax.numpy as jnp
from jax import lax
from jax.experimental import pallas as pl
from jax.experimental.pallas import tpu as pltpu
from jax.experimental.pallas import tpu_sc as plsc

N_NODES = 10000
D_IN = 128
D_HID = 128
D_OUT = 64

NC, NS = 2, 16
NW = NC * NS
CH = 128
ROWS = 10112
STRIPE = ROWS // NS

TC_BLK = 2000
TC_GRID = N_NODES // TC_BLK


def _sc_mesh():
    return plsc.VectorSubcoreMesh(
        core_axis_name="c", subcore_axis_name="s", num_cores=NC, num_subcores=NS
    )


def _sc_degree(dst_p, kch):
    @functools.partial(
        pl.kernel,
        out_type=jax.ShapeDtypeStruct((NC, ROWS, 16), jnp.float32),
        mesh=_sc_mesh(),
        scratch_types=[
            pltpu.VMEM_SHARED((ROWS, 16), jnp.float32),
            pltpu.VMEM((kch, CH), jnp.int32),
            pltpu.VMEM((CH, 16), jnp.float32),
            pltpu.VMEM((STRIPE, 16), jnp.float32),
        ],
    )
    def k(dst_hbm, out_hbm, acc_sh, idx_v, ones_v, zero_v):
        cid = lax.axis_index("c")
        sid = lax.axis_index("s")
        wid = cid * NS + sid
        base = sid * STRIPE

        def fill_ones(i, carry):
            ones_v[i, :] = jnp.ones((16,), jnp.float32)
            return carry

        lax.fori_loop(0, CH, fill_ones, 0)

        def fill_zero(i, carry):
            zero_v[i, :] = jnp.zeros((16,), jnp.float32)
            return carry

        lax.fori_loop(0, STRIPE, fill_zero, 0)

        pltpu.sync_copy(zero_v, acc_sh.at[pl.ds(base, STRIPE)])
        plsc.subcore_barrier()

        pltpu.sync_copy(dst_hbm.at[wid], idx_v)

        def step(j, carry):
            pltpu.sync_copy(ones_v, acc_sh.at[idx_v.at[j]], add=True)
            return carry

        lax.fori_loop(0, kch, step, 0)
        plsc.subcore_barrier()
        pltpu.sync_copy(
            acc_sh.at[pl.ds(base, STRIPE)], out_hbm.at[cid, pl.ds(base, STRIPE)]
        )

    return k(dst_p)


def _sc_aggregate(y, src_p, dst_p, kch):
    ig = 16
    assert kch % ig == 0

    @functools.partial(
        pl.kernel,
        out_type=jax.ShapeDtypeStruct((NC, ROWS, D_HID), jnp.float32),
        mesh=_sc_mesh(),
        scratch_types=[
            pltpu.VMEM_SHARED((ROWS, D_HID), jnp.float32),
            pltpu.VMEM((ig, CH), jnp.int32),
            pltpu.VMEM((ig, CH), jnp.int32),
            pltpu.VMEM((CH, D_HID), jnp.float32),
            pltpu.VMEM((CH, D_HID), jnp.float32),
            pltpu.SemaphoreType.DMA,
            pltpu.SemaphoreType.DMA,
        ],
    )
    def k(y_hbm, src_hbm, dst_hbm, out_hbm, acc_sh, sidx, didx, rows_v, rows_w, sem, sem2):
        cid = lax.axis_index("c")
        sid = lax.axis_index("s")
        wid = cid * NS + sid
        base = sid * STRIPE

        def fill_zero(i, carry):
            for c in range(D_HID // 16):
                rows_v[i, pl.ds(c * 16, 16)] = jnp.zeros((16,), jnp.float32)
            return carry

        lax.fori_loop(0, CH, fill_zero, 0)

        for off in range(0, STRIPE - CH + 1, CH):
            pltpu.sync_copy(rows_v, acc_sh.at[pl.ds(base + off, CH)])
        rem = STRIPE % CH
        if rem:
            pltpu.sync_copy(
                rows_v.at[pl.ds(0, rem)], acc_sh.at[pl.ds(base + STRIPE - rem, rem)]
            )
        plsc.subcore_barrier()

        bufs = ((rows_v, sem), (rows_w, sem2))

        def group(g, carry):
            pltpu.sync_copy(src_hbm.at[wid, pl.ds(g * ig, ig)], sidx)
            pltpu.sync_copy(dst_hbm.at[wid, pl.ds(g * ig, ig)], didx)
            pltpu.async_copy(y_hbm.at[sidx.at[0]], rows_v, sem)
            pltpu.async_copy(y_hbm.at[sidx.at[1]], rows_w, sem2)

            def pair(h, carry2):
                for b in range(2):
                    j = 2 * h + b
                    rv, sm = bufs[b]
                    pltpu.make_async_copy(y_hbm.at[sidx.at[j]], rv, sm).wait()
                    pltpu.sync_copy(rv, acc_sh.at[didx.at[j]], add=True)

                    @pl.when(j + 2 < ig)
                    def _prefetch():
                        pltpu.async_copy(y_hbm.at[sidx.at[j + 2]], rv, sm)

                return carry2

            lax.fori_loop(0, ig // 2, pair, 0)
            return carry

        lax.fori_loop(0, kch // ig, group, 0)
        plsc.subcore_barrier()
        pltpu.sync_copy(
            acc_sh.at[pl.ds(base, STRIPE)], out_hbm.at[cid, pl.ds(base, STRIPE)]
        )

    return k(y, src_p, dst_p)


def _tc_y_body(degp_ref, x_ref, wg_ref, y_ref):
    dsum = degp_ref[0] + degp_ref[1]
    dinv = lax.rsqrt(dsum[:, 0:1] + 1.0)
    xw = jnp.dot(x_ref[...], wg_ref[...], preferred_element_type=jnp.float32)
    y_ref[...] = xw * dinv


def _tc_head_body(degp_ref, p_ref, y_ref, bg_ref, wl_ref, bl_ref, out_ref):
    dsum = degp_ref[0] + degp_ref[1]
    dinv = lax.rsqrt(dsum[:, 0:1] + 1.0)
    agg = p_ref[0] + p_ref[1] + y_ref[...]
    h = jnp.maximum(agg * dinv + bg_ref[...], 0.0)
    out_ref[...] = (
        jnp.dot(h, wl_ref[...], preferred_element_type=jnp.float32) + bl_ref[...]
    )


def kernel(x, edge_index, W_g, b_g, W_l, b_l):
    src = edge_index[0].astype(jnp.int32)
    dst = edge_index[1].astype(jnp.int32)
    n_edges = src.shape[0]
    per_w = n_edges // NW
    kch = -(-per_w // CH)
    kch = -(-kch // 16) * 16
    per_w_pad = kch * CH

    pad = per_w_pad - per_w
    src_p = jnp.concatenate(
        [src.reshape(NW, per_w), jnp.zeros((NW, pad), jnp.int32)], axis=1
    ).reshape(NW, kch, CH)
    garbage = N_NODES + (jnp.arange(pad, dtype=jnp.int32) % 16)
    dst_p = jnp.concatenate(
        [dst.reshape(NW, per_w), jnp.broadcast_to(garbage, (NW, pad))], axis=1
    ).reshape(NW, kch, CH)

    degp = _sc_degree(dst_p, kch)

    y = pl.pallas_call(
        _tc_y_body,
        grid=(TC_GRID,),
        in_specs=[
            pl.BlockSpec((NC, TC_BLK, 16), lambda i: (0, i, 0)),
            pl.BlockSpec((TC_BLK, D_IN), lambda i: (i, 0)),
            pl.BlockSpec((D_IN, D_HID), lambda i: (0, 0)),
        ],
        out_specs=pl.BlockSpec((TC_BLK, D_HID), lambda i: (i, 0)),
        out_shape=jax.ShapeDtypeStruct((N_NODES, D_HID), jnp.float32),
    )(degp, x, W_g)

    p = _sc_aggregate(y, src_p, dst_p, kch)

    out = pl.pallas_call(
        _tc_head_body,
        grid=(TC_GRID,),
        in_specs=[
            pl.BlockSpec((NC, TC_BLK, 16), lambda i: (0, i, 0)),
            pl.BlockSpec((NC, TC_BLK, D_HID), lambda i: (0, i, 0)),
            pl.BlockSpec((TC_BLK, D_HID), lambda i: (i, 0)),
            pl.BlockSpec((1, D_HID), lambda i: (0, 0)),
            pl.BlockSpec((D_HID, D_OUT), lambda i: (0, 0)),
            pl.BlockSpec((1, D_OUT), lambda i: (0, 0)),
        ],
        out_specs=pl.BlockSpec((TC_BLK, D_OUT), lambda i: (i, 0)),
        out_shape=jax.ShapeDtypeStruct((N_NODES, D_OUT), jnp.float32),
    )(degp, p, y, b_g.reshape(1, D_HID), W_l, b_l.reshape(1, D_OUT))

    return out


# speedup vs baseline: 16.1096x; 1.0048x over previous
import functools

import jax
import jax.numpy as jnp
from jax import lax
from jax.experimental import pallas as pl
from jax.experimental.pallas import tpu as pltpu
from jax.experimental.pallas import tpu_sc as plsc

N_NODES = 10000
D_IN = 128
D_HID = 128
D_OUT = 64

NC, NS = 2, 16
NW = NC * NS
CH = 64
ROWS = 10112
STRIPE = ROWS // NS

TC_BLK = 2000
TC_GRID = N_NODES // TC_BLK


def _sc_mesh():
    return plsc.VectorSubcoreMesh(
        core_axis_name="c", subcore_axis_name="s", num_cores=NC, num_subcores=NS
    )


def _sc_degree(dst_p, kch):
    @functools.partial(
        pl.kernel,
        out_type=jax.ShapeDtypeStruct((NC, ROWS, 16), jnp.float32),
        mesh=_sc_mesh(),
        scratch_types=[
            pltpu.VMEM_SHARED((ROWS, 16), jnp.float32),
            pltpu.VMEM((kch, CH), jnp.int32),
            pltpu.VMEM((CH, 16), jnp.float32),
            pltpu.VMEM((STRIPE, 16), jnp.float32),
        ],
    )
    def k(dst_hbm, out_hbm, acc_sh, idx_v, ones_v, zero_v):
        cid = lax.axis_index("c")
        sid = lax.axis_index("s")
        wid = cid * NS + sid
        base = sid * STRIPE

        def fill_ones(i, carry):
            ones_v[i, :] = jnp.ones((16,), jnp.float32)
            return carry

        lax.fori_loop(0, CH, fill_ones, 0)

        def fill_zero(i, carry):
            zero_v[i, :] = jnp.zeros((16,), jnp.float32)
            return carry

        lax.fori_loop(0, STRIPE, fill_zero, 0)

        pltpu.sync_copy(zero_v, acc_sh.at[pl.ds(base, STRIPE)])
        plsc.subcore_barrier()

        pltpu.sync_copy(dst_hbm.at[wid], idx_v)

        def step(j, carry):
            pltpu.sync_copy(ones_v, acc_sh.at[idx_v.at[j]], add=True)
            return carry

        lax.fori_loop(0, kch, step, 0)
        plsc.subcore_barrier()
        pltpu.sync_copy(
            acc_sh.at[pl.ds(base, STRIPE)], out_hbm.at[cid, pl.ds(base, STRIPE)]
        )

    return k(dst_p)


def _sc_aggregate(y, src_p, dst_p, kch):
    ig = 32
    assert kch % ig == 0

    @functools.partial(
        pl.kernel,
        out_type=jax.ShapeDtypeStruct((NC, ROWS, D_HID), jnp.float32),
        mesh=_sc_mesh(),
        scratch_types=[
            pltpu.VMEM_SHARED((ROWS, D_HID), jnp.float32),
            pltpu.VMEM((ig, CH), jnp.int32),
            pltpu.VMEM((ig, CH), jnp.int32),
            pltpu.VMEM((CH, D_HID), jnp.float32),
            pltpu.VMEM((CH, D_HID), jnp.float32),
            pltpu.VMEM((CH, D_HID), jnp.float32),
            pltpu.VMEM((CH, D_HID), jnp.float32),
            pltpu.SemaphoreType.DMA,
            pltpu.SemaphoreType.DMA,
            pltpu.SemaphoreType.DMA,
            pltpu.SemaphoreType.DMA,
        ],
    )
    def k(y_hbm, src_hbm, dst_hbm, out_hbm, acc_sh, sidx, didx,
          r0, r1, r2, r3, s0, s1, s2, s3):
        rows_v = r0
        cid = lax.axis_index("c")
        sid = lax.axis_index("s")
        wid = cid * NS + sid
        base = sid * STRIPE

        def fill_zero(i, carry):
            for c in range(D_HID // 16):
                rows_v[i, pl.ds(c * 16, 16)] = jnp.zeros((16,), jnp.float32)
            return carry

        lax.fori_loop(0, CH, fill_zero, 0)

        for off in range(0, STRIPE - CH + 1, CH):
            pltpu.sync_copy(rows_v, acc_sh.at[pl.ds(base + off, CH)])
        rem = STRIPE % CH
        if rem:
            pltpu.sync_copy(
                rows_v.at[pl.ds(0, rem)], acc_sh.at[pl.ds(base + STRIPE - rem, rem)]
            )
        plsc.subcore_barrier()

        bufs = ((r0, s0), (r1, s1), (r2, s2), (r3, s3))
        nbuf = len(bufs)

        def group(g, carry):
            pltpu.sync_copy(src_hbm.at[wid, pl.ds(g * ig, ig)], sidx)
            pltpu.sync_copy(dst_hbm.at[wid, pl.ds(g * ig, ig)], didx)
            for b in range(nbuf):
                pltpu.async_copy(y_hbm.at[sidx.at[b]], bufs[b][0], bufs[b][1])

            def quad(q, carry2):
                for b in range(nbuf):
                    j = nbuf * q + b
                    rv, sm = bufs[b]
                    pltpu.make_async_copy(y_hbm.at[sidx.at[j]], rv, sm).wait()
                    pltpu.sync_copy(rv, acc_sh.at[didx.at[j]], add=True)

                    @pl.when(j + nbuf < ig)
                    def _prefetch():
                        pltpu.async_copy(y_hbm.at[sidx.at[j + nbuf]], rv, sm)

                return carry2

            lax.fori_loop(0, ig // nbuf, quad, 0)
            return carry

        lax.fori_loop(0, kch // ig, group, 0)
        plsc.subcore_barrier()
        pltpu.sync_copy(
            acc_sh.at[pl.ds(base, STRIPE)], out_hbm.at[cid, pl.ds(base, STRIPE)]
        )

    return k(y, src_p, dst_p)


def _tc_y_body(degp_ref, x_ref, wg_ref, y_ref):
    dsum = degp_ref[0] + degp_ref[1]
    dinv = lax.rsqrt(dsum[:, 0:1] + 1.0)
    xw = jnp.dot(x_ref[...], wg_ref[...], preferred_element_type=jnp.float32)
    y_ref[...] = xw * dinv


def _tc_head_body(degp_ref, p_ref, y_ref, bg_ref, wl_ref, bl_ref, out_ref):
    dsum = degp_ref[0] + degp_ref[1]
    dinv = lax.rsqrt(dsum[:, 0:1] + 1.0)
    agg = p_ref[0] + p_ref[1] + y_ref[...]
    h = jnp.maximum(agg * dinv + bg_ref[...], 0.0)
    out_ref[...] = (
        jnp.dot(h, wl_ref[...], preferred_element_type=jnp.float32) + bl_ref[...]
    )


def kernel(x, edge_index, W_g, b_g, W_l, b_l):
    src = edge_index[0].astype(jnp.int32)
    dst = edge_index[1].astype(jnp.int32)
    n_edges = src.shape[0]
    per_w = n_edges // NW
    kch = -(-per_w // CH)
    kch = -(-kch // 32) * 32
    per_w_pad = kch * CH

    pad = per_w_pad - per_w
    src_p = jnp.concatenate(
        [src.reshape(NW, per_w), jnp.zeros((NW, pad), jnp.int32)], axis=1
    ).reshape(NW, kch, CH)
    garbage = N_NODES + (jnp.arange(pad, dtype=jnp.int32) % 16)
    dst_p = jnp.concatenate(
        [dst.reshape(NW, per_w), jnp.broadcast_to(garbage, (NW, pad))], axis=1
    ).reshape(NW, kch, CH)

    degp = _sc_degree(dst_p, kch)

    y = pl.pallas_call(
        _tc_y_body,
        grid=(TC_GRID,),
        in_specs=[
            pl.BlockSpec((NC, TC_BLK, 16), lambda i: (0, i, 0)),
            pl.BlockSpec((TC_BLK, D_IN), lambda i: (i, 0)),
            pl.BlockSpec((D_IN, D_HID), lambda i: (0, 0)),
        ],
        out_specs=pl.BlockSpec((TC_BLK, D_HID), lambda i: (i, 0)),
        out_shape=jax.ShapeDtypeStruct((N_NODES, D_HID), jnp.float32),
    )(degp, x, W_g)

    p = _sc_aggregate(y, src_p, dst_p, kch)

    out = pl.pallas_call(
        _tc_head_body,
        grid=(TC_GRID,),
        in_specs=[
            pl.BlockSpec((NC, TC_BLK, 16), lambda i: (0, i, 0)),
            pl.BlockSpec((NC, TC_BLK, D_HID), lambda i: (0, i, 0)),
            pl.BlockSpec((TC_BLK, D_HID), lambda i: (i, 0)),
            pl.BlockSpec((1, D_HID), lambda i: (0, 0)),
            pl.BlockSpec((D_HID, D_OUT), lambda i: (0, 0)),
            pl.BlockSpec((1, D_OUT), lambda i: (0, 0)),
        ],
        out_specs=pl.BlockSpec((TC_BLK, D_OUT), lambda i: (i, 0)),
        out_shape=jax.ShapeDtypeStruct((N_NODES, D_OUT), jnp.float32),
    )(degp, p, y, b_g.reshape(1, D_HID), W_l, b_l.reshape(1, D_OUT))

    return out
